# baseline (speedup 1.0000x reference)
import functools

import jax
import jax.numpy as jnp
from jax import lax
from jax.experimental import pallas as pl
from jax.experimental.pallas import tpu as pltpu
from jax.experimental.pallas import tpu_sc as plsc

E = 64
TOPK = 2
D = 1024
F = 1024
B = 128
CHUNK = 256


def _router_body(x_ref, wg_ref, w_ref, pos_ref, be_ref, nact_ref):
    x = x_ref[...]
    wg = wg_ref[...]
    N = x.shape[0]
    P = TOPK * N
    NBLK = P // B + E
    logits = lax.dot_general(x, wg, (((1,), (1,)), ((), ())),
                             preferred_element_type=jnp.float32)
    scores = jax.nn.sigmoid(logits)
    iota_e = lax.broadcasted_iota(jnp.int32, (N, E), 1)
    s1 = jnp.max(scores, axis=1, keepdims=True)
    i1 = jnp.min(jnp.where(scores == s1, iota_e, E), axis=1, keepdims=True)
    masked = jnp.where(iota_e == i1, -jnp.inf, scores)
    s2 = jnp.max(masked, axis=1, keepdims=True)
    i2 = jnp.min(jnp.where(masked == s2, iota_e, E), axis=1, keepdims=True)
    sw = s1 + s2
    w_ref[...] = jnp.broadcast_to(
        jnp.concatenate([s1 / sw, s2 / sw], axis=0), (P, 16))
    oh1 = (iota_e == i1).astype(jnp.float32)
    oh2 = (iota_e == i2).astype(jnp.float32)
    onehot = jnp.concatenate([oh1, oh2], axis=0)
    ri = lax.broadcasted_iota(jnp.int32, (CHUNK, CHUNK), 0)
    ci = lax.broadcasted_iota(jnp.int32, (CHUNK, CHUNK), 1)
    L = (ci <= ri).astype(jnp.float32)
    parts = []
    carry = jnp.zeros((1, E), jnp.float32)
    for b in range(P // CHUNK):
        seg = onehot[b * CHUNK:(b + 1) * CHUNK]
        c = jnp.dot(L, seg, preferred_element_type=jnp.float32) + carry
        parts.append(c)
        carry = c[CHUNK - 1:CHUNK, :]
    cum = jnp.concatenate(parts, axis=0)
    rank = jnp.sum(cum * onehot, axis=1, keepdims=True) - 1.0
    total = carry
    nblk = jnp.floor((total + (B - 1)) * (1.0 / B))
    re_ = lax.broadcasted_iota(jnp.int32, (E, E), 0)
    ce_ = lax.broadcasted_iota(jnp.int32, (E, E), 1)
    U = (re_ < ce_).astype(jnp.float32)
    excl = jnp.dot(nblk, U, preferred_element_type=jnp.float32)
    row_off = excl * B
    pos = jnp.sum(onehot * row_off, axis=1, keepdims=True) + rank
    pos_ref[...] = pos.astype(jnp.int32)
    ident = (re_ == ce_).astype(jnp.float32)
    excl_col = lax.dot_general(ident, excl, (((1,), (1,)), ((), ())),
                               preferred_element_type=jnp.float32)
    blk_i = lax.broadcasted_iota(jnp.int32, (E, NBLK), 1)
    le = (excl_col.astype(jnp.int32) <= blk_i).astype(jnp.float32)
    be = jnp.sum(le, axis=0, keepdims=True) - 1.0
    be_ref[...] = be.astype(jnp.int32)
    nact = (excl + nblk)[:, E - 1:E]
    nact_ref[...] = nact.astype(jnp.int32)


def _router(x, w_gate):
    N = x.shape[0]
    P = TOPK * N
    NBLK = P // B + E
    return pl.pallas_call(
        _router_body,
        out_shape=[
            jax.ShapeDtypeStruct((P, 16), jnp.float32),
            jax.ShapeDtypeStruct((P, 1), jnp.int32),
            jax.ShapeDtypeStruct((1, NBLK), jnp.int32),
            jax.ShapeDtypeStruct((1, 1), jnp.int32),
        ],
    )(x, w_gate)


def _moe_body(be_ref, nact_ref, xs_ref, wgu_ref, wd_ref, ys_ref):
    del be_ref

    @pl.when(pl.program_id(0) < nact_ref[0])
    def _():
        gu = jnp.dot(xs_ref[...], wgu_ref[0],
                     preferred_element_type=jnp.float32)
        g = gu[:, :F]
        u = gu[:, F:]
        h = (g * jax.nn.sigmoid(g)) * u
        ys_ref[...] = jnp.dot(h, wd_ref[0], preferred_element_type=jnp.float32)


def _moe(xs_pad, w_gate_up, w_down, be, nact):
    NBLK = be.shape[0]
    P_PAD = xs_pad.shape[0]
    grid_spec = pltpu.PrefetchScalarGridSpec(
        num_scalar_prefetch=2,
        grid=(NBLK,),
        in_specs=[
            pl.BlockSpec((B, D), lambda i, be, na: (i, 0)),
            pl.BlockSpec((1, D, 2 * F), lambda i, be, na: (be[i], 0, 0)),
            pl.BlockSpec((1, F, D), lambda i, be, na: (be[i], 0, 0)),
        ],
        out_specs=pl.BlockSpec((B, D), lambda i, be, na: (i, 0)),
    )
    return pl.pallas_call(
        _moe_body,
        grid_spec=grid_spec,
        out_shape=jax.ShapeDtypeStruct((P_PAD, D), jnp.float32),
    )(be, nact, xs_pad, w_gate_up, w_down)


_NC = 2
_NS = 16
_NW = _NC * _NS


def _dispatch(x, pos, p_pad):
    N, d = x.shape
    P = pos.shape[0]
    sub = 64
    per_w = P // _NW
    mesh = plsc.VectorSubcoreMesh(core_axis_name="c", subcore_axis_name="s")

    @functools.partial(
        pl.kernel,
        out_type=jax.ShapeDtypeStruct((p_pad, d), jnp.float32),
        mesh=mesh,
        scratch_types=[
            pltpu.VMEM((sub,), jnp.int32),
            pltpu.VMEM((sub,), jnp.int32),
            pltpu.VMEM((sub, d), jnp.float32),
            pltpu.SemaphoreType.DMA,
        ],
    )
    def disp(x_hbm, pos_hbm, xs_hbm, tok_v, idx_v, rows_v, sem):
        wid = lax.axis_index("s") * _NC + lax.axis_index("c")
        base = wid * per_w
        for s in range(per_w // sub):
            jb = base + s * sub
            pltpu.sync_copy(pos_hbm.at[pl.ds(jb, sub)], idx_v)
            for c in range(sub // 16):
                j16 = lax.iota(jnp.int32, 16) + (jb + c * 16)
                tok_v[pl.ds(c * 16, 16)] = jnp.where(j16 >= N, j16 - N, j16)
            pltpu.async_copy(x_hbm.at[tok_v], rows_v, sem).wait()
            pltpu.async_copy(rows_v, xs_hbm.at[idx_v], sem).wait()

    return disp(x, pos)


def _combine(ys_pad, pos, w_rep, N):
    d = ys_pad.shape[1]
    csub = 32
    per_w = N // _NW
    mesh = plsc.VectorSubcoreMesh(core_axis_name="c", subcore_axis_name="s")

    @functools.partial(
        pl.kernel,
        out_type=jax.ShapeDtypeStruct((N, d), jnp.float32),
        mesh=mesh,
        scratch_types=[
            pltpu.VMEM((csub,), jnp.int32),
            pltpu.VMEM((csub,), jnp.int32),
            pltpu.VMEM((csub, 16), jnp.float32),
            pltpu.VMEM((csub, 16), jnp.float32),
            pltpu.VMEM((csub, d), jnp.float32),
            pltpu.VMEM((csub, d), jnp.float32),
            pltpu.VMEM((csub, d), jnp.float32),
            pltpu.SemaphoreType.DMA,
        ],
    )
    def comb(ys_hbm, pos_hbm, w_hbm, out_hbm,
             i0_v, i1_v, w0_v, w1_v, r0_v, r1_v, o_v, sem):
        wid = lax.axis_index("s") * _NC + lax.axis_index("c")
        tbase = wid * per_w
        for s in range(per_w // csub):
            tb = tbase + s * csub
            pltpu.sync_copy(pos_hbm.at[pl.ds(tb, csub)], i0_v)
            pltpu.sync_copy(pos_hbm.at[pl.ds(N + tb, csub)], i1_v)
            pltpu.sync_copy(w_hbm.at[pl.ds(tb, csub)], w0_v)
            pltpu.sync_copy(w_hbm.at[pl.ds(N + tb, csub)], w1_v)
            pltpu.async_copy(ys_hbm.at[i0_v], r0_v, sem).wait()
            pltpu.async_copy(ys_hbm.at[i1_v], r1_v, sem).wait()

            def body_row(r, carry):
                w0c = w0_v[r]
                w1c = w1_v[r]

                def body_c(ci, c2):
                    c = pl.multiple_of(lax.shift_left(ci, 4), 16)
                    o_v[r, pl.ds(c, 16)] = (r0_v[r, pl.ds(c, 16)] * w0c
                                            + r1_v[r, pl.ds(c, 16)] * w1c)
                    return c2

                lax.fori_loop(0, d // 16, body_c, 0, unroll=8)
                return carry

            lax.fori_loop(0, csub, body_row, 0)
            pltpu.sync_copy(o_v, out_hbm.at[pl.ds(tb, csub)])

    return comb(ys_pad, pos, w_rep)


def kernel(hidden_states, w_gate, w_gate_up, w_down):
    orig_shape = hidden_states.shape
    x = hidden_states.reshape(-1, D)
    N = x.shape[0]
    P = TOPK * N
    NBLK = P // B + E
    P_PAD = NBLK * B
    w_rep, pos_col, be_row, nact_arr = _router(x, w_gate)
    pos = pos_col.reshape(P)
    be = be_row.reshape(NBLK)
    nact = nact_arr.reshape(1)
    xs_pad = _dispatch(x, pos, P_PAD)
    ys_pad = _moe(xs_pad, w_gate_up, w_down, be, nact)
    out = _combine(ys_pad, pos, w_rep, N)
    return out.reshape(orig_shape)


# speedup vs baseline: 1.0272x; 1.0272x over previous
import functools

import jax
import jax.numpy as jnp
from jax import lax
from jax.experimental import pallas as pl
from jax.experimental.pallas import tpu as pltpu
from jax.experimental.pallas import tpu_sc as plsc

E = 64
TOPK = 2
D = 1024
F = 1024
B = 128
CHUNK = 256


def _router_body(x_ref, wg_ref, w_ref, pos_ref, be_ref, nact_ref):
    x = x_ref[...]
    wg = wg_ref[...]
    N = x.shape[0]
    P = TOPK * N
    NBLK = P // B + E
    logits = lax.dot_general(x, wg, (((1,), (1,)), ((), ())),
                             preferred_element_type=jnp.float32)
    scores = jax.nn.sigmoid(logits)
    iota_e = lax.broadcasted_iota(jnp.int32, (N, E), 1)
    s1 = jnp.max(scores, axis=1, keepdims=True)
    i1 = jnp.min(jnp.where(scores == s1, iota_e, E), axis=1, keepdims=True)
    masked = jnp.where(iota_e == i1, -jnp.inf, scores)
    s2 = jnp.max(masked, axis=1, keepdims=True)
    i2 = jnp.min(jnp.where(masked == s2, iota_e, E), axis=1, keepdims=True)
    sw = s1 + s2
    w_ref[...] = jnp.broadcast_to(
        jnp.concatenate([s1 / sw, s2 / sw], axis=0), (P, 16))
    oh1 = (iota_e == i1).astype(jnp.float32)
    oh2 = (iota_e == i2).astype(jnp.float32)
    onehot = jnp.concatenate([oh1, oh2], axis=0)
    ri = lax.broadcasted_iota(jnp.int32, (CHUNK, CHUNK), 0)
    ci = lax.broadcasted_iota(jnp.int32, (CHUNK, CHUNK), 1)
    L = (ci <= ri).astype(jnp.float32)
    parts = []
    carry = jnp.zeros((1, E), jnp.float32)
    for b in range(P // CHUNK):
        seg = onehot[b * CHUNK:(b + 1) * CHUNK]
        c = jnp.dot(L, seg, preferred_element_type=jnp.float32) + carry
        parts.append(c)
        carry = c[CHUNK - 1:CHUNK, :]
    cum = jnp.concatenate(parts, axis=0)
    rank = jnp.sum(cum * onehot, axis=1, keepdims=True) - 1.0
    total = carry
    nblk = jnp.floor((total + (B - 1)) * (1.0 / B))
    re_ = lax.broadcasted_iota(jnp.int32, (E, E), 0)
    ce_ = lax.broadcasted_iota(jnp.int32, (E, E), 1)
    U = (re_ < ce_).astype(jnp.float32)
    excl = jnp.dot(nblk, U, preferred_element_type=jnp.float32)
    row_off = excl * B
    pos = jnp.sum(onehot * row_off, axis=1, keepdims=True) + rank
    pos_ref[...] = pos.astype(jnp.int32)
    ident = (re_ == ce_).astype(jnp.float32)
    excl_col = lax.dot_general(ident, excl, (((1,), (1,)), ((), ())),
                               preferred_element_type=jnp.float32)
    blk_i = lax.broadcasted_iota(jnp.int32, (E, NBLK), 1)
    le = (excl_col.astype(jnp.int32) <= blk_i).astype(jnp.float32)
    be = jnp.sum(le, axis=0, keepdims=True) - 1.0
    be_ref[...] = be.astype(jnp.int32)
    nact = (excl + nblk)[:, E - 1:E]
    nact_ref[...] = nact.astype(jnp.int32)


def _router(x, w_gate):
    N = x.shape[0]
    P = TOPK * N
    NBLK = P // B + E
    return pl.pallas_call(
        _router_body,
        out_shape=[
            jax.ShapeDtypeStruct((P, 16), jnp.float32),
            jax.ShapeDtypeStruct((P, 1), jnp.int32),
            jax.ShapeDtypeStruct((1, NBLK), jnp.int32),
            jax.ShapeDtypeStruct((1, 1), jnp.int32),
        ],
    )(x, w_gate)


def _moe_body(be_ref, nact_ref, xs_ref, wgu_ref, wd_ref, ys_ref):
    del be_ref

    @pl.when(pl.program_id(0) < nact_ref[0])
    def _():
        gu = jnp.dot(xs_ref[...], wgu_ref[0],
                     preferred_element_type=jnp.float32)
        g = gu[:, :F]
        u = gu[:, F:]
        h = (g * jax.nn.sigmoid(g)) * u
        ys_ref[...] = jnp.dot(h, wd_ref[0], preferred_element_type=jnp.float32)


def _moe(xs_pad, w_gate_up, w_down, be, nact):
    NBLK = be.shape[0]
    P_PAD = xs_pad.shape[0]
    grid_spec = pltpu.PrefetchScalarGridSpec(
        num_scalar_prefetch=2,
        grid=(NBLK,),
        in_specs=[
            pl.BlockSpec((B, D),
                         lambda i, be, na: (jnp.where(i < na[0], i, 0), 0)),
            pl.BlockSpec((1, D, 2 * F), lambda i, be, na: (be[i], 0, 0)),
            pl.BlockSpec((1, F, D), lambda i, be, na: (be[i], 0, 0)),
        ],
        out_specs=pl.BlockSpec((B, D), lambda i, be, na: (i, 0)),
    )
    return pl.pallas_call(
        _moe_body,
        grid_spec=grid_spec,
        out_shape=jax.ShapeDtypeStruct((P_PAD, D), jnp.float32),
    )(be, nact, xs_pad, w_gate_up, w_down)


_NC = 2
_NS = 16
_NW = _NC * _NS


def _dispatch(x, pos, p_pad):
    N, d = x.shape
    P = pos.shape[0]
    sub = 64
    per_w = P // _NW
    mesh = plsc.VectorSubcoreMesh(core_axis_name="c", subcore_axis_name="s")

    @functools.partial(
        pl.kernel,
        out_type=jax.ShapeDtypeStruct((p_pad, d), jnp.float32),
        mesh=mesh,
        scratch_types=[
            pltpu.VMEM((sub,), jnp.int32),
            pltpu.VMEM((sub,), jnp.int32),
            pltpu.VMEM((sub, d), jnp.float32),
            pltpu.SemaphoreType.DMA,
        ],
    )
    def disp(x_hbm, pos_hbm, xs_hbm, tok_v, idx_v, rows_v, sem):
        wid = lax.axis_index("s") * _NC + lax.axis_index("c")
        base = wid * per_w
        for s in range(per_w // sub):
            jb = base + s * sub
            pltpu.sync_copy(pos_hbm.at[pl.ds(jb, sub)], idx_v)
            for c in range(sub // 16):
                j16 = lax.iota(jnp.int32, 16) + (jb + c * 16)
                tok_v[pl.ds(c * 16, 16)] = jnp.where(j16 >= N, j16 - N, j16)
            pltpu.async_copy(x_hbm.at[tok_v], rows_v, sem).wait()
            pltpu.async_copy(rows_v, xs_hbm.at[idx_v], sem).wait()

    return disp(x, pos)


def _combine(ys_pad, pos, w_rep, N):
    d = ys_pad.shape[1]
    csub = 32
    per_w = N // _NW
    mesh = plsc.VectorSubcoreMesh(core_axis_name="c", subcore_axis_name="s")

    @functools.partial(
        pl.kernel,
        out_type=jax.ShapeDtypeStruct((N, d), jnp.float32),
        mesh=mesh,
        scratch_types=[
            pltpu.VMEM((csub,), jnp.int32),
            pltpu.VMEM((csub,), jnp.int32),
            pltpu.VMEM((csub, 16), jnp.float32),
            pltpu.VMEM((csub, 16), jnp.float32),
            pltpu.VMEM((csub, d), jnp.float32),
            pltpu.VMEM((csub, d), jnp.float32),
            pltpu.VMEM((csub, d), jnp.float32),
            pltpu.SemaphoreType.DMA,
        ],
    )
    def comb(ys_hbm, pos_hbm, w_hbm, out_hbm,
             i0_v, i1_v, w0_v, w1_v, r0_v, r1_v, o_v, sem):
        wid = lax.axis_index("s") * _NC + lax.axis_index("c")
        tbase = wid * per_w
        for s in range(per_w // csub):
            tb = tbase + s * csub
            pltpu.sync_copy(pos_hbm.at[pl.ds(tb, csub)], i0_v)
            pltpu.sync_copy(pos_hbm.at[pl.ds(N + tb, csub)], i1_v)
            pltpu.sync_copy(w_hbm.at[pl.ds(tb, csub)], w0_v)
            pltpu.sync_copy(w_hbm.at[pl.ds(N + tb, csub)], w1_v)
            pltpu.async_copy(ys_hbm.at[i0_v], r0_v, sem).wait()
            pltpu.async_copy(ys_hbm.at[i1_v], r1_v, sem).wait()

            def body_row(r, carry):
                w0c = w0_v[r]
                w1c = w1_v[r]

                def body_c(ci, c2):
                    c = pl.multiple_of(lax.shift_left(ci, 4), 16)
                    o_v[r, pl.ds(c, 16)] = (r0_v[r, pl.ds(c, 16)] * w0c
                                            + r1_v[r, pl.ds(c, 16)] * w1c)
                    return c2

                lax.fori_loop(0, d // 16, body_c, 0, unroll=8)
                return carry

            lax.fori_loop(0, csub, body_row, 0)
            pltpu.sync_copy(o_v, out_hbm.at[pl.ds(tb, csub)])

    return comb(ys_pad, pos, w_rep)


def kernel(hidden_states, w_gate, w_gate_up, w_down):
    orig_shape = hidden_states.shape
    x = hidden_states.reshape(-1, D)
    N = x.shape[0]
    P = TOPK * N
    NBLK = P // B + E
    P_PAD = NBLK * B
    w_rep, pos_col, be_row, nact_arr = _router(x, w_gate)
    pos = pos_col.reshape(P)
    be = be_row.reshape(NBLK)
    nact = nact_arr.reshape(1)
    xs_pad = _dispatch(x, pos, P_PAD)
    ys_pad = _moe(xs_pad, w_gate_up, w_down, be, nact)
    out = _combine(ys_pad, pos, w_rep, N)
    return out.reshape(orig_shape)
